# baseline (speedup 1.0000x reference)
import functools

import jax
import jax.numpy as jnp
from jax import lax
from jax.experimental import pallas as pl
from jax.experimental.pallas import tpu as pltpu
from jax.experimental.pallas import tpu_sc as plsc

N = 10000
E = 320000
D_OBS = 128
HID = 256
MID = 128
LSTM_D = 128
OUT = 128
RFM = 256
LAST = 128
LH = 128
ACT = 16

W = 144
WT = 144
NS = 16
NC = 2
K = 64
EPAD = 1536
EP = E + EPAD
EPT = EP // NS
NCHUNK = EPT // K
ZROW = 4 * N
DROW = N
NP = 10112
RPT = NP // NS

R1 = 2000
R2 = 2000

_HI = jax.lax.Precision.DEFAULT


def _sigmoid(x):
    return 1.0 / (1.0 + jnp.exp(-x))


def _tc1_body(obs, h, c, w1at, b1a, w1bt, b1b, wiht, whht, bg, wm1t, bm1,
              wst, wdt, be_, h_o, c_o, nf_o, a_o, b_o):
    x = jnp.maximum(jnp.dot(obs[...], w1at[...], precision=_HI) + b1a[...], 0.0)
    x = jnp.dot(x, w1bt[...], precision=_HI) + b1b[...]
    g = (jnp.dot(x, wiht[...], precision=_HI)
         + jnp.dot(h[...], whht[...], precision=_HI) + bg[...])
    i_g = _sigmoid(g[:, :LSTM_D])
    f_g = _sigmoid(g[:, LSTM_D:2 * LSTM_D])
    g_g = jnp.tanh(g[:, 2 * LSTM_D:3 * LSTM_D])
    o_g = _sigmoid(g[:, 3 * LSTM_D:])
    c_new = f_g * c[...] + i_g * g_g
    h_new = o_g * jnp.tanh(c_new)
    nf = jnp.dot(jnp.maximum(h_new, 0.0), wm1t[...], precision=_HI) + bm1[...]
    a_full = jnp.dot(nf, wst[...], precision=_HI)
    b_full = jnp.dot(nf, wdt[...], precision=_HI) + be_[...]
    h_o[...] = h_new
    c_o[...] = c_new
    nf_o[...] = nf
    rows = a_full.shape[0]
    colid = lax.broadcasted_iota(jnp.int32, (rows, 16), 1)
    pad_one = jnp.where(colid == 0, 1.0, 0.0).astype(jnp.float32)
    pad_zero = jnp.zeros((rows, 16), jnp.float32)
    a_o[0] = jnp.concatenate([a_full[:, :128], pad_one], axis=1)
    a_o[1] = jnp.concatenate([a_full[:, 128:], pad_one], axis=1)
    b_o[0] = jnp.concatenate([b_full[:, :128], pad_zero], axis=1)
    b_o[1] = jnp.concatenate([b_full[:, 128:], pad_zero], axis=1)


def _tc1(obs, h, c, w1at, b1a, w1bt, b1b, wiht, whht, bg, wm1t, bm1, wst, wdt, be_):
    grid = (N // R1,)
    row = lambda i: (i, 0)
    fixed = lambda i: (0, 0)
    out3 = lambda i: (0, i, 0)
    return pl.pallas_call(
        _tc1_body,
        grid=grid,
        in_specs=[
            pl.BlockSpec((R1, D_OBS), row),
            pl.BlockSpec((R1, LSTM_D), row),
            pl.BlockSpec((R1, LSTM_D), row),
            pl.BlockSpec((D_OBS, HID), fixed),
            pl.BlockSpec((1, HID), fixed),
            pl.BlockSpec((HID, MID), fixed),
            pl.BlockSpec((1, MID), fixed),
            pl.BlockSpec((MID, 4 * LSTM_D), fixed),
            pl.BlockSpec((LSTM_D, 4 * LSTM_D), fixed),
            pl.BlockSpec((1, 4 * LSTM_D), fixed),
            pl.BlockSpec((LSTM_D, OUT), fixed),
            pl.BlockSpec((1, OUT), fixed),
            pl.BlockSpec((OUT, RFM), fixed),
            pl.BlockSpec((OUT, RFM), fixed),
            pl.BlockSpec((1, RFM), fixed),
        ],
        out_specs=[
            pl.BlockSpec((R1, LSTM_D), row),
            pl.BlockSpec((R1, LSTM_D), row),
            pl.BlockSpec((R1, OUT), row),
            pl.BlockSpec((NC, R1, WT), out3),
            pl.BlockSpec((NC, R1, WT), out3),
        ],
        out_shape=[
            jax.ShapeDtypeStruct((N, LSTM_D), jnp.float32),
            jax.ShapeDtypeStruct((N, LSTM_D), jnp.float32),
            jax.ShapeDtypeStruct((N, OUT), jnp.float32),
            jax.ShapeDtypeStruct((NC, N, WT), jnp.float32),
            jax.ShapeDtypeStruct((NC, N, WT), jnp.float32),
        ],
    )(obs, h, c, w1at, b1a, w1bt, b1b, wiht, whht, bg, wm1t, bm1, wst, wdt, be_)


def _sc_body(t_tab, gidx, dst1, zeros_hbm, m_out,
             gbuf0, gbuf1, sbuf0, sbuf1, ebuf0, ebuf1,
             acc, semg0, semg1, sems0, sems1, semi):
    c = lax.axis_index("c")
    s = lax.axis_index("s")
    rb = s * RPT
    pltpu.sync_copy(zeros_hbm.at[pl.ds(rb, RPT)], acc.at[pl.ds(rb, RPT)])
    plsc.subcore_barrier()

    GBUF = [gbuf0, gbuf1]
    SBUF = [sbuf0, sbuf1]
    EBUF = [ebuf0, ebuf1]
    SEMG = [semg0, semg1]
    SEMS = [sems0, sems1]

    gbase = c * (NS * NCHUNK) + s * NCHUNK
    ebase = s * EPT

    def fetch_idx(k, slot):
        pltpu.async_copy(gidx.at[pl.ds((gbase + k) * 2 * K, 2 * K)],
                         GBUF[slot], semi)
        pltpu.async_copy(dst1.at[pl.ds(ebase + k * K, K)], SBUF[slot], semi)

    def wait_idx(slot):
        pltpu.make_async_copy(gidx.at[pl.ds(0, 2 * K)], GBUF[slot], semi).wait()
        pltpu.make_async_copy(dst1.at[pl.ds(0, K)], SBUF[slot], semi).wait()

    def start_g(slot):
        pltpu.async_copy(t_tab.at[GBUF[slot]], EBUF[slot], SEMG[slot])

    def wait_g(slot):
        pltpu.make_async_copy(t_tab.at[pl.ds(0, 2 * K)], EBUF[slot],
                              SEMG[slot]).wait()

    fetch_idx(0, 0)
    wait_idx(0)
    start_g(0)

    @pl.loop(0, NCHUNK // 2)
    def _pair(i):
        for b in range(2):
            p, q = b, 1 - b
            k = 2 * i + b

            def drain_q():
                pltpu.make_async_copy(EBUF[q].at[pl.ds(0, K)],
                                      acc.at[pl.ds(0, K)], SEMS[q]).wait()

            def prefetch():
                fetch_idx(k + 1, q)
                wait_idx(q)
                start_g(q)

            if b == 0:
                pl.when(i > 0)(drain_q)
                prefetch()
            else:
                drain_q()
                pl.when(i < NCHUNK // 2 - 1)(prefetch)

            wait_g(p)

            @pl.loop(0, K, unroll=2)
            def _relu(r):
                for j in range(WT // 16):
                    va = EBUF[p][r, pl.ds(j * 16, 16)]
                    vb = EBUF[p][K + r, pl.ds(j * 16, 16)]
                    EBUF[p][r, pl.ds(j * 16, 16)] = jnp.maximum(va + vb, 0.0)

            pltpu.async_copy(EBUF[p].at[pl.ds(0, K)], acc.at[SBUF[p]],
                             SEMS[p], add=True)

    pltpu.make_async_copy(EBUF[1].at[pl.ds(0, K)], acc.at[pl.ds(0, K)],
                          SEMS[1]).wait()
    plsc.subcore_barrier()
    pltpu.sync_copy(acc.at[pl.ds(rb, RPT)], m_out.at[pl.ds(c * NP + rb, RPT)])


def _sc_edge(t_tab, gidx, dst1, zeros_hbm):
    mesh = plsc.VectorSubcoreMesh(core_axis_name="c", subcore_axis_name="s")
    f = pl.kernel(
        _sc_body,
        out_type=jax.ShapeDtypeStruct((NC * NP, W), jnp.float32),
        mesh=mesh,
        scratch_types=[
            pltpu.VMEM((2 * K,), jnp.int32),
            pltpu.VMEM((2 * K,), jnp.int32),
            pltpu.VMEM((K,), jnp.int32),
            pltpu.VMEM((K,), jnp.int32),
            pltpu.VMEM((2 * K, WT), jnp.float32),
            pltpu.VMEM((2 * K, WT), jnp.float32),
            pltpu.VMEM_SHARED((NP, W), jnp.float32),
            pltpu.SemaphoreType.DMA,
            pltpu.SemaphoreType.DMA,
            pltpu.SemaphoreType.DMA,
            pltpu.SemaphoreType.DMA,
            pltpu.SemaphoreType.DMA,
        ],
        compiler_params=pltpu.CompilerParams(use_tc_tiling_on_sc=False),
    )
    return f(t_tab, gidx, dst1, zeros_hbm)


def _tc2_body(m0, m1, nf, we2at, we2bt, be2r, wnat, wnbt, bnr, wn2t, bn2r,
              wrot, bror, wro2t, bro2r, q_o):
    h_agg = (jnp.dot(m0[...], we2at[...], precision=_HI)
             + jnp.dot(m1[...], we2bt[...], precision=_HI)
             + m0[:, 128:129] * be2r[...])
    nh = jnp.maximum(jnp.dot(nf[...], wnat[...], precision=_HI)
                     + jnp.dot(h_agg, wnbt[...], precision=_HI) + bnr[...], 0.0)
    node_out = jnp.dot(nh, wn2t[...], precision=_HI) + bn2r[...]
    qh = jnp.maximum(jnp.dot(node_out, wrot[...], precision=_HI) + bror[...], 0.0)
    q_o[...] = jnp.dot(qh, wro2t[...], precision=_HI) + bro2r[...]


def _tc2(m0, m1, nf, we2at, we2bt, be2r, wnat, wnbt, bnr, wn2t, bn2r,
         wrot, bror, wro2t, bro2r):
    grid = (N // R2,)
    row = lambda i: (i, 0)
    fixed = lambda i: (0, 0)
    return pl.pallas_call(
        _tc2_body,
        grid=grid,
        in_specs=[
            pl.BlockSpec((R2, W), row),
            pl.BlockSpec((R2, W), row),
            pl.BlockSpec((R2, OUT), row),
            pl.BlockSpec((W, LAST), fixed),
            pl.BlockSpec((W, LAST), fixed),
            pl.BlockSpec((1, LAST), fixed),
            pl.BlockSpec((OUT, RFM), fixed),
            pl.BlockSpec((LAST, RFM), fixed),
            pl.BlockSpec((1, RFM), fixed),
            pl.BlockSpec((RFM, LAST), fixed),
            pl.BlockSpec((1, LAST), fixed),
            pl.BlockSpec((LAST, LH), fixed),
            pl.BlockSpec((1, LH), fixed),
            pl.BlockSpec((LH, ACT), fixed),
            pl.BlockSpec((1, ACT), fixed),
        ],
        out_specs=[pl.BlockSpec((R2, ACT), row)],
        out_shape=[jax.ShapeDtypeStruct((N, ACT), jnp.float32)],
    )(m0, m1, nf, we2at, we2bt, be2r, wnat, wnbt, bnr, wn2t, bn2r,
      wrot, bror, wro2t, bro2r)


def kernel(obs, h0, c0, edge_index, W1a, b1a, W1b, b1b, W_ih, b_ih, W_hh, b_hh,
           Wm1, bm1, We, be, We2, be2, Wn, bn, Wn2, bn2, Wu, bu, Wu2, bu2,
           Wro, bro, Wro2, bro2):
    h = h0[0]
    c = c0[0]
    w1at = W1a.T
    w1bt = W1b.T
    wiht = W_ih.T
    whht = W_hh.T
    bg = (b_ih + b_hh)[None, :]
    wm1t = Wm1.T
    wst = We[:, :OUT].T
    wdt = We[:, OUT:].T

    h_new, c_new, nf, a_ext, b_ext = _tc1(
        obs, h, c, w1at, b1a[None, :], w1bt, b1b[None, :], wiht, whht, bg,
        wm1t, bm1[None, :], wst, wdt, be[None, :])

    t_tab = jnp.concatenate([
        jnp.stack([a_ext, b_ext], axis=1).reshape(2 * NC * N, WT),
        jnp.zeros((8, WT), jnp.float32)], axis=0)
    src = edge_index[0]
    dst = edge_index[1]
    padv = jnp.full((EPAD,), ZROW, jnp.int32)
    s0 = jnp.concatenate([src, padv]).reshape(EP // K, K)
    d0 = jnp.concatenate([dst + N, padv]).reshape(EP // K, K)
    s1 = jnp.concatenate([src + 2 * N, padv]).reshape(EP // K, K)
    d1 = jnp.concatenate([dst + 3 * N, padv]).reshape(EP // K, K)
    g0 = jnp.stack([s0, d0], axis=1)
    g1 = jnp.stack([s1, d1], axis=1)
    gidx = jnp.stack([g0, g1], axis=0).reshape(-1)
    dstp = jnp.concatenate([dst, jnp.full((EPAD,), DROW, jnp.int32)])
    zeros_hbm = jnp.zeros((NP, W), jnp.float32)

    m = _sc_edge(t_tab, gidx, dstp, zeros_hbm)

    p0 = jnp.zeros((W, LAST), jnp.float32).at[:128].set(We2[:, :LAST].T)
    p1 = jnp.zeros((W, LAST), jnp.float32).at[:128].set(We2[:, LAST:].T)

    q = _tc2(m[:N], m[NP:NP + N], nf,
             p0, p1, be2[None, :],
             Wn[:, :OUT].T, Wn[:, OUT:].T, bn[None, :],
             Wn2.T, bn2[None, :], Wro.T, bro[None, :], Wro2.T, bro2[None, :])[0]

    return q, h_new[None], c_new[None]


# speedup vs baseline: 1.1412x; 1.1412x over previous
import functools

import jax
import jax.numpy as jnp
from jax import lax
from jax.experimental import pallas as pl
from jax.experimental.pallas import tpu as pltpu
from jax.experimental.pallas import tpu_sc as plsc

N = 10000
E = 320000
D_OBS = 128
HID = 256
MID = 128
LSTM_D = 128
OUT = 128
RFM = 256
LAST = 128
LH = 128
ACT = 16

W = 144
WT = 144
NS = 16
NC = 2
K = 80
EPAD = 2560
EP = E + EPAD
EPT = EP // NS
NCHUNK = EPT // K
ZROW = NC * N
DROW = N
NP = 10112
RPT = NP // NS

R1 = 2000
R2 = 2000

_HI = jax.lax.Precision.DEFAULT


def _sigmoid(x):
    return 1.0 / (1.0 + jnp.exp(-x))


def _tc1_body(obs, h, c, w1at, b1a, w1bt, b1b, wiht, whht, bg, wm1t, bm1,
              wst, wdt, be_, h_o, c_o, nf_o, a_o, b_o):
    x = jnp.maximum(jnp.dot(obs[...], w1at[...], precision=_HI) + b1a[...], 0.0)
    x = jnp.dot(x, w1bt[...], precision=_HI) + b1b[...]
    g = (jnp.dot(x, wiht[...], precision=_HI)
         + jnp.dot(h[...], whht[...], precision=_HI) + bg[...])
    i_g = _sigmoid(g[:, :LSTM_D])
    f_g = _sigmoid(g[:, LSTM_D:2 * LSTM_D])
    g_g = jnp.tanh(g[:, 2 * LSTM_D:3 * LSTM_D])
    o_g = _sigmoid(g[:, 3 * LSTM_D:])
    c_new = f_g * c[...] + i_g * g_g
    h_new = o_g * jnp.tanh(c_new)
    nf = jnp.dot(jnp.maximum(h_new, 0.0), wm1t[...], precision=_HI) + bm1[...]
    a_full = jnp.dot(nf, wst[...], precision=_HI)
    b_full = jnp.dot(nf, wdt[...], precision=_HI) + be_[...]
    h_o[...] = h_new
    c_o[...] = c_new
    nf_o[...] = nf
    rows = a_full.shape[0]
    colid = lax.broadcasted_iota(jnp.int32, (rows, 16), 1)
    pad_one = jnp.where(colid == 0, 1.0, 0.0).astype(jnp.float32)
    pad_zero = jnp.zeros((rows, 16), jnp.float32)
    a_o[0] = jnp.concatenate([a_full[:, :128], pad_one], axis=1)
    a_o[1] = jnp.concatenate([a_full[:, 128:], pad_one], axis=1)
    b_o[0] = jnp.concatenate([b_full[:, :128], pad_zero], axis=1)
    b_o[1] = jnp.concatenate([b_full[:, 128:], pad_zero], axis=1)


def _tc1(obs, h, c, w1at, b1a, w1bt, b1b, wiht, whht, bg, wm1t, bm1, wst, wdt, be_):
    grid = (N // R1,)
    row = lambda i: (i, 0)
    fixed = lambda i: (0, 0)
    out3 = lambda i: (0, i, 0)
    return pl.pallas_call(
        _tc1_body,
        grid=grid,
        in_specs=[
            pl.BlockSpec((R1, D_OBS), row),
            pl.BlockSpec((R1, LSTM_D), row),
            pl.BlockSpec((R1, LSTM_D), row),
            pl.BlockSpec((D_OBS, HID), fixed),
            pl.BlockSpec((1, HID), fixed),
            pl.BlockSpec((HID, MID), fixed),
            pl.BlockSpec((1, MID), fixed),
            pl.BlockSpec((MID, 4 * LSTM_D), fixed),
            pl.BlockSpec((LSTM_D, 4 * LSTM_D), fixed),
            pl.BlockSpec((1, 4 * LSTM_D), fixed),
            pl.BlockSpec((LSTM_D, OUT), fixed),
            pl.BlockSpec((1, OUT), fixed),
            pl.BlockSpec((OUT, RFM), fixed),
            pl.BlockSpec((OUT, RFM), fixed),
            pl.BlockSpec((1, RFM), fixed),
        ],
        out_specs=[
            pl.BlockSpec((R1, LSTM_D), row),
            pl.BlockSpec((R1, LSTM_D), row),
            pl.BlockSpec((R1, OUT), row),
            pl.BlockSpec((NC, R1, WT), out3),
            pl.BlockSpec((NC, R1, WT), out3),
        ],
        out_shape=[
            jax.ShapeDtypeStruct((N, LSTM_D), jnp.float32),
            jax.ShapeDtypeStruct((N, LSTM_D), jnp.float32),
            jax.ShapeDtypeStruct((N, OUT), jnp.float32),
            jax.ShapeDtypeStruct((NC, N, WT), jnp.float32),
            jax.ShapeDtypeStruct((NC, N, WT), jnp.float32),
        ],
    )(obs, h, c, w1at, b1a, w1bt, b1b, wiht, whht, bg, wm1t, bm1, wst, wdt, be_)


def _sc_body(a_tab, b_tab, gidx, dst1, zeros_hbm, m_out,
             gbuf0, gbuf1, gbuf2, sbuf0, sbuf1, sbuf2,
             abuf0, abuf1, abuf2, acc,
             semg0, semg1, semg2, sems0, sems1, sems2, semi):
    c = lax.axis_index("c")
    s = lax.axis_index("s")
    rb = s * RPT
    pltpu.sync_copy(zeros_hbm.at[pl.ds(rb, RPT)], acc.at[pl.ds(rb, RPT)])
    plsc.subcore_barrier()

    GBUF = [gbuf0, gbuf1, gbuf2]
    SBUF = [sbuf0, sbuf1, sbuf2]
    ABUF = [abuf0, abuf1, abuf2]
    SEMG = [semg0, semg1, semg2]
    SEMS = [sems0, sems1, sems2]

    gbase = c * (NS * NCHUNK) + s * NCHUNK
    ebase = s * EPT

    def fetch_idx(k, slot):
        pltpu.async_copy(gidx.at[pl.ds((gbase + k) * 2 * K, 2 * K)],
                         GBUF[slot], semi)
        pltpu.async_copy(dst1.at[pl.ds(ebase + k * K, K)], SBUF[slot], semi)

    def wait_idx(slot):
        pltpu.make_async_copy(gidx.at[pl.ds(0, 2 * K)], GBUF[slot], semi).wait()
        pltpu.make_async_copy(dst1.at[pl.ds(0, K)], SBUF[slot], semi).wait()

    def start_a(slot):
        pltpu.async_copy(a_tab.at[GBUF[slot].at[pl.ds(0, K)]], ABUF[slot],
                         SEMG[slot])

    def start_b_add(slot):
        pltpu.async_copy(b_tab.at[GBUF[slot].at[pl.ds(K, K)]], ABUF[slot],
                         SEMG[slot], add=True)

    def wait_g(slot):
        pltpu.make_async_copy(a_tab.at[pl.ds(0, K)], ABUF[slot],
                              SEMG[slot]).wait()

    fetch_idx(0, 0)
    wait_idx(0)
    start_a(0)
    fetch_idx(1, 1)
    wait_idx(1)
    start_a(1)
    wait_g(0)
    start_b_add(0)
    fetch_idx(2, 2)
    wait_idx(2)
    start_a(2)

    @pl.loop(0, NCHUNK // 3)
    def _tri(i):
        for b in range(3):
            p = b
            pn = (b + 1) % 3
            pd = (b + 2) % 3
            k = 3 * i + b

            wait_g(p)

            @pl.loop(0, K, unroll=2)
            def _relu(r):
                for j in range(WT // 16):
                    v = ABUF[p][r, pl.ds(j * 16, 16)]
                    ABUF[p][r, pl.ds(j * 16, 16)] = jnp.maximum(v, 0.0)

            pltpu.async_copy(ABUF[p], acc.at[SBUF[p]], SEMS[p], add=True)

            def adv():
                wait_g(pn)
                start_b_add(pn)
            if b < 2:
                adv()
            else:
                pl.when(i < NCHUNK // 3 - 1)(adv)

            def drain():
                pltpu.make_async_copy(ABUF[pd], acc.at[pl.ds(0, K)],
                                      SEMS[pd]).wait()

            def prep():
                fetch_idx(k + 2, pd)
                wait_idx(pd)
                start_a(pd)

            if b == 0:
                def drain_prep():
                    drain()
                    prep()
                pl.when(i > 0)(drain_prep)
            else:
                drain()
                pl.when(i < NCHUNK // 3 - 1)(prep)

    pltpu.make_async_copy(ABUF[2], acc.at[pl.ds(0, K)], SEMS[2]).wait()
    plsc.subcore_barrier()
    pltpu.sync_copy(acc.at[pl.ds(rb, RPT)], m_out.at[pl.ds(c * NP + rb, RPT)])


def _sc_edge(a_tab, b_tab, gidx, dst1, zeros_hbm):
    mesh = plsc.VectorSubcoreMesh(core_axis_name="c", subcore_axis_name="s")
    f = pl.kernel(
        _sc_body,
        out_type=jax.ShapeDtypeStruct((NC * NP, W), jnp.float32),
        mesh=mesh,
        scratch_types=[
            pltpu.VMEM((2 * K,), jnp.int32),
            pltpu.VMEM((2 * K,), jnp.int32),
            pltpu.VMEM((2 * K,), jnp.int32),
            pltpu.VMEM((K,), jnp.int32),
            pltpu.VMEM((K,), jnp.int32),
            pltpu.VMEM((K,), jnp.int32),
            pltpu.VMEM((K, WT), jnp.float32),
            pltpu.VMEM((K, WT), jnp.float32),
            pltpu.VMEM((K, WT), jnp.float32),
            pltpu.VMEM_SHARED((NP, W), jnp.float32),
            pltpu.SemaphoreType.DMA,
            pltpu.SemaphoreType.DMA,
            pltpu.SemaphoreType.DMA,
            pltpu.SemaphoreType.DMA,
            pltpu.SemaphoreType.DMA,
            pltpu.SemaphoreType.DMA,
            pltpu.SemaphoreType.DMA,
        ],
        compiler_params=pltpu.CompilerParams(use_tc_tiling_on_sc=False),
    )
    return f(a_tab, b_tab, gidx, dst1, zeros_hbm)


def _tc2_body(m0, m1, nf, we2at, we2bt, be2r, wnat, wnbt, bnr, wn2t, bn2r,
              wrot, bror, wro2t, bro2r, q_o):
    h_agg = (jnp.dot(m0[...], we2at[...], precision=_HI)
             + jnp.dot(m1[...], we2bt[...], precision=_HI)
             + m0[:, 128:129] * be2r[...])
    nh = jnp.maximum(jnp.dot(nf[...], wnat[...], precision=_HI)
                     + jnp.dot(h_agg, wnbt[...], precision=_HI) + bnr[...], 0.0)
    node_out = jnp.dot(nh, wn2t[...], precision=_HI) + bn2r[...]
    qh = jnp.maximum(jnp.dot(node_out, wrot[...], precision=_HI) + bror[...], 0.0)
    q_o[...] = jnp.dot(qh, wro2t[...], precision=_HI) + bro2r[...]


def _tc2(m0, m1, nf, we2at, we2bt, be2r, wnat, wnbt, bnr, wn2t, bn2r,
         wrot, bror, wro2t, bro2r):
    grid = (N // R2,)
    row = lambda i: (i, 0)
    fixed = lambda i: (0, 0)
    return pl.pallas_call(
        _tc2_body,
        grid=grid,
        in_specs=[
            pl.BlockSpec((R2, W), row),
            pl.BlockSpec((R2, W), row),
            pl.BlockSpec((R2, OUT), row),
            pl.BlockSpec((W, LAST), fixed),
            pl.BlockSpec((W, LAST), fixed),
            pl.BlockSpec((1, LAST), fixed),
            pl.BlockSpec((OUT, RFM), fixed),
            pl.BlockSpec((LAST, RFM), fixed),
            pl.BlockSpec((1, RFM), fixed),
            pl.BlockSpec((RFM, LAST), fixed),
            pl.BlockSpec((1, LAST), fixed),
            pl.BlockSpec((LAST, LH), fixed),
            pl.BlockSpec((1, LH), fixed),
            pl.BlockSpec((LH, ACT), fixed),
            pl.BlockSpec((1, ACT), fixed),
        ],
        out_specs=[pl.BlockSpec((R2, ACT), row)],
        out_shape=[jax.ShapeDtypeStruct((N, ACT), jnp.float32)],
    )(m0, m1, nf, we2at, we2bt, be2r, wnat, wnbt, bnr, wn2t, bn2r,
      wrot, bror, wro2t, bro2r)


def kernel(obs, h0, c0, edge_index, W1a, b1a, W1b, b1b, W_ih, b_ih, W_hh, b_hh,
           Wm1, bm1, We, be, We2, be2, Wn, bn, Wn2, bn2, Wu, bu, Wu2, bu2,
           Wro, bro, Wro2, bro2):
    h = h0[0]
    c = c0[0]
    w1at = W1a.T
    w1bt = W1b.T
    wiht = W_ih.T
    whht = W_hh.T
    bg = (b_ih + b_hh)[None, :]
    wm1t = Wm1.T
    wst = We[:, :OUT].T
    wdt = We[:, OUT:].T

    h_new, c_new, nf, a_ext, b_ext = _tc1(
        obs, h, c, w1at, b1a[None, :], w1bt, b1b[None, :], wiht, whht, bg,
        wm1t, bm1[None, :], wst, wdt, be[None, :])

    zrow = jnp.zeros((8, WT), jnp.float32)
    a_tab = jnp.concatenate([a_ext.reshape(NC * N, WT), zrow], axis=0)
    b_tab = jnp.concatenate([b_ext.reshape(NC * N, WT), zrow], axis=0)
    src = edge_index[0]
    dst = edge_index[1]
    padv = jnp.full((EPAD,), ZROW, jnp.int32)
    s0 = jnp.concatenate([src, padv]).reshape(EP // K, K)
    d0 = jnp.concatenate([dst, padv]).reshape(EP // K, K)
    s1 = jnp.concatenate([src + N, padv]).reshape(EP // K, K)
    d1 = jnp.concatenate([dst + N, padv]).reshape(EP // K, K)
    g0 = jnp.stack([s0, d0], axis=1)
    g1 = jnp.stack([s1, d1], axis=1)
    gidx = jnp.stack([g0, g1], axis=0).reshape(-1)
    dstp = jnp.concatenate([dst, jnp.full((EPAD,), DROW, jnp.int32)])
    zeros_hbm = jnp.zeros((NP, W), jnp.float32)

    m = _sc_edge(a_tab, b_tab, gidx, dstp, zeros_hbm)

    p0 = jnp.zeros((W, LAST), jnp.float32).at[:128].set(We2[:, :LAST].T)
    p1 = jnp.zeros((W, LAST), jnp.float32).at[:128].set(We2[:, LAST:].T)

    q = _tc2(m[:N], m[NP:NP + N], nf,
             p0, p1, be2[None, :],
             Wn[:, :OUT].T, Wn[:, OUT:].T, bn[None, :],
             Wn2.T, bn2[None, :], Wro.T, bro[None, :], Wro2.T, bro2[None, :])[0]

    return q, h_new[None], c_new[None]


# speedup vs baseline: 1.1599x; 1.0164x over previous
import functools

import jax
import jax.numpy as jnp
from jax import lax
from jax.experimental import pallas as pl
from jax.experimental.pallas import tpu as pltpu
from jax.experimental.pallas import tpu_sc as plsc

N = 10000
E = 320000
D_OBS = 128
HID = 256
MID = 128
LSTM_D = 128
OUT = 128
RFM = 256
LAST = 128
LH = 128
ACT = 16

W = 144
WT = 144
NS = 16
NC = 2
K = 128
EPAD = 3584
EP = E + EPAD
EPT = EP // NS
NCHUNK = EPT // K
ZROW = NC * N
DROW = N
NP = 10112
RPT = NP // NS

R1 = 2000
R2 = 2000

_HI = jax.lax.Precision.DEFAULT


def _sigmoid(x):
    return 1.0 / (1.0 + jnp.exp(-x))


def _tc1_body(obs, h, c, w1at, b1a, w1bt, b1b, wiht, whht, bg, wm1t, bm1,
              wst, wdt, be_, h_o, c_o, nf_o, a_o, b_o):
    x = jnp.maximum(jnp.dot(obs[...], w1at[...], precision=_HI) + b1a[...], 0.0)
    x = jnp.dot(x, w1bt[...], precision=_HI) + b1b[...]
    g = (jnp.dot(x, wiht[...], precision=_HI)
         + jnp.dot(h[...], whht[...], precision=_HI) + bg[...])
    i_g = _sigmoid(g[:, :LSTM_D])
    f_g = _sigmoid(g[:, LSTM_D:2 * LSTM_D])
    g_g = jnp.tanh(g[:, 2 * LSTM_D:3 * LSTM_D])
    o_g = _sigmoid(g[:, 3 * LSTM_D:])
    c_new = f_g * c[...] + i_g * g_g
    h_new = o_g * jnp.tanh(c_new)
    nf = jnp.dot(jnp.maximum(h_new, 0.0), wm1t[...], precision=_HI) + bm1[...]
    a_full = jnp.dot(nf, wst[...], precision=_HI)
    b_full = jnp.dot(nf, wdt[...], precision=_HI) + be_[...]
    h_o[...] = h_new
    c_o[...] = c_new
    nf_o[...] = nf
    rows = a_full.shape[0]
    colid = lax.broadcasted_iota(jnp.int32, (rows, 16), 1)
    pad_one = jnp.where(colid == 0, 1.0, 0.0).astype(jnp.float32)
    pad_zero = jnp.zeros((rows, 16), jnp.float32)
    a_o[0] = jnp.concatenate([a_full[:, :128], pad_one], axis=1)
    a_o[1] = jnp.concatenate([a_full[:, 128:], pad_one], axis=1)
    b_o[0] = jnp.concatenate([b_full[:, :128], pad_zero], axis=1)
    b_o[1] = jnp.concatenate([b_full[:, 128:], pad_zero], axis=1)


def _tc1(obs, h, c, w1at, b1a, w1bt, b1b, wiht, whht, bg, wm1t, bm1, wst, wdt, be_):
    grid = (N // R1,)
    row = lambda i: (i, 0)
    fixed = lambda i: (0, 0)
    out3 = lambda i: (0, i, 0)
    return pl.pallas_call(
        _tc1_body,
        grid=grid,
        in_specs=[
            pl.BlockSpec((R1, D_OBS), row),
            pl.BlockSpec((R1, LSTM_D), row),
            pl.BlockSpec((R1, LSTM_D), row),
            pl.BlockSpec((D_OBS, HID), fixed),
            pl.BlockSpec((1, HID), fixed),
            pl.BlockSpec((HID, MID), fixed),
            pl.BlockSpec((1, MID), fixed),
            pl.BlockSpec((MID, 4 * LSTM_D), fixed),
            pl.BlockSpec((LSTM_D, 4 * LSTM_D), fixed),
            pl.BlockSpec((1, 4 * LSTM_D), fixed),
            pl.BlockSpec((LSTM_D, OUT), fixed),
            pl.BlockSpec((1, OUT), fixed),
            pl.BlockSpec((OUT, RFM), fixed),
            pl.BlockSpec((OUT, RFM), fixed),
            pl.BlockSpec((1, RFM), fixed),
        ],
        out_specs=[
            pl.BlockSpec((R1, LSTM_D), row),
            pl.BlockSpec((R1, LSTM_D), row),
            pl.BlockSpec((R1, OUT), row),
            pl.BlockSpec((NC, R1, WT), out3),
            pl.BlockSpec((NC, R1, WT), out3),
        ],
        out_shape=[
            jax.ShapeDtypeStruct((N, LSTM_D), jnp.float32),
            jax.ShapeDtypeStruct((N, LSTM_D), jnp.float32),
            jax.ShapeDtypeStruct((N, OUT), jnp.float32),
            jax.ShapeDtypeStruct((NC, N, WT), jnp.float32),
            jax.ShapeDtypeStruct((NC, N, WT), jnp.float32),
        ],
    )(obs, h, c, w1at, b1a, w1bt, b1b, wiht, whht, bg, wm1t, bm1, wst, wdt, be_)


def _sc_body(a_tab, b_tab, gidx, dst1, zeros_hbm, m_out,
             gbuf0, gbuf1, sbuf0, sbuf1, abuf0, abuf1, acc,
             semg0, semg1, sems0, sems1, semi):
    c = lax.axis_index("c")
    s = lax.axis_index("s")
    rb = s * RPT
    pltpu.sync_copy(zeros_hbm.at[pl.ds(rb, RPT)], acc.at[pl.ds(rb, RPT)])
    plsc.subcore_barrier()

    GBUF = [gbuf0, gbuf1]
    SBUF = [sbuf0, sbuf1]
    ABUF = [abuf0, abuf1]
    SEMG = [semg0, semg1]
    SEMS = [sems0, sems1]

    gbase = c * (NS * NCHUNK) + s * NCHUNK
    ebase = s * EPT

    def fetch_idx(k, slot):
        pltpu.async_copy(gidx.at[pl.ds((gbase + k) * 2 * K, 2 * K)],
                         GBUF[slot], semi)
        pltpu.async_copy(dst1.at[pl.ds(ebase + k * K, K)], SBUF[slot], semi)

    def wait_idx(slot):
        pltpu.make_async_copy(gidx.at[pl.ds(0, 2 * K)], GBUF[slot], semi).wait()
        pltpu.make_async_copy(dst1.at[pl.ds(0, K)], SBUF[slot], semi).wait()

    def start_a(slot):
        pltpu.async_copy(a_tab.at[GBUF[slot].at[pl.ds(0, K)]], ABUF[slot],
                         SEMG[slot])

    def start_b_add(slot):
        pltpu.async_copy(b_tab.at[GBUF[slot].at[pl.ds(K, K)]], ABUF[slot],
                         SEMG[slot], add=True)

    def wait_g(slot):
        pltpu.make_async_copy(a_tab.at[pl.ds(0, K)], ABUF[slot],
                              SEMG[slot]).wait()

    fetch_idx(0, 0)
    wait_idx(0)
    start_a(0)

    @pl.loop(0, NCHUNK // 2)
    def _pair(i):
        for b in range(2):
            p, q = b, 1 - b
            k = 2 * i + b

            wait_g(p)
            start_b_add(p)

            def drain_q():
                pltpu.make_async_copy(ABUF[q], acc.at[pl.ds(0, K)],
                                      SEMS[q]).wait()

            def prefetch():
                fetch_idx(k + 1, q)
                wait_idx(q)
                start_a(q)

            if b == 0:
                pl.when(i > 0)(drain_q)
                prefetch()
            else:
                drain_q()
                pl.when(i < NCHUNK // 2 - 1)(prefetch)

            wait_g(p)

            @pl.loop(0, K, unroll=2)
            def _relu(r):
                for j in range(WT // 16):
                    v = ABUF[p][r, pl.ds(j * 16, 16)]
                    ABUF[p][r, pl.ds(j * 16, 16)] = jnp.maximum(v, 0.0)

            pltpu.async_copy(ABUF[p], acc.at[SBUF[p]], SEMS[p], add=True)

    pltpu.make_async_copy(ABUF[1], acc.at[pl.ds(0, K)], SEMS[1]).wait()
    plsc.subcore_barrier()
    pltpu.sync_copy(acc.at[pl.ds(rb, RPT)], m_out.at[pl.ds(c * NP + rb, RPT)])


def _sc_edge(a_tab, b_tab, gidx, dst1, zeros_hbm):
    mesh = plsc.VectorSubcoreMesh(core_axis_name="c", subcore_axis_name="s")
    f = pl.kernel(
        _sc_body,
        out_type=jax.ShapeDtypeStruct((NC * NP, W), jnp.float32),
        mesh=mesh,
        scratch_types=[
            pltpu.VMEM((2 * K,), jnp.int32),
            pltpu.VMEM((2 * K,), jnp.int32),
            pltpu.VMEM((K,), jnp.int32),
            pltpu.VMEM((K,), jnp.int32),
            pltpu.VMEM((K, WT), jnp.float32),
            pltpu.VMEM((K, WT), jnp.float32),
            pltpu.VMEM_SHARED((NP, W), jnp.float32),
            pltpu.SemaphoreType.DMA,
            pltpu.SemaphoreType.DMA,
            pltpu.SemaphoreType.DMA,
            pltpu.SemaphoreType.DMA,
            pltpu.SemaphoreType.DMA,
        ],
        compiler_params=pltpu.CompilerParams(use_tc_tiling_on_sc=False),
    )
    return f(a_tab, b_tab, gidx, dst1, zeros_hbm)


def _tc2_body(m0, m1, nf, we2at, we2bt, be2r, wnat, wnbt, bnr, wn2t, bn2r,
              wrot, bror, wro2t, bro2r, q_o):
    h_agg = (jnp.dot(m0[...], we2at[...], precision=_HI)
             + jnp.dot(m1[...], we2bt[...], precision=_HI)
             + m0[:, 128:129] * be2r[...])
    nh = jnp.maximum(jnp.dot(nf[...], wnat[...], precision=_HI)
                     + jnp.dot(h_agg, wnbt[...], precision=_HI) + bnr[...], 0.0)
    node_out = jnp.dot(nh, wn2t[...], precision=_HI) + bn2r[...]
    qh = jnp.maximum(jnp.dot(node_out, wrot[...], precision=_HI) + bror[...], 0.0)
    q_o[...] = jnp.dot(qh, wro2t[...], precision=_HI) + bro2r[...]


def _tc2(m0, m1, nf, we2at, we2bt, be2r, wnat, wnbt, bnr, wn2t, bn2r,
         wrot, bror, wro2t, bro2r):
    grid = (N // R2,)
    row = lambda i: (i, 0)
    fixed = lambda i: (0, 0)
    return pl.pallas_call(
        _tc2_body,
        grid=grid,
        in_specs=[
            pl.BlockSpec((R2, W), row),
            pl.BlockSpec((R2, W), row),
            pl.BlockSpec((R2, OUT), row),
            pl.BlockSpec((W, LAST), fixed),
            pl.BlockSpec((W, LAST), fixed),
            pl.BlockSpec((1, LAST), fixed),
            pl.BlockSpec((OUT, RFM), fixed),
            pl.BlockSpec((LAST, RFM), fixed),
            pl.BlockSpec((1, RFM), fixed),
            pl.BlockSpec((RFM, LAST), fixed),
            pl.BlockSpec((1, LAST), fixed),
            pl.BlockSpec((LAST, LH), fixed),
            pl.BlockSpec((1, LH), fixed),
            pl.BlockSpec((LH, ACT), fixed),
            pl.BlockSpec((1, ACT), fixed),
        ],
        out_specs=[pl.BlockSpec((R2, ACT), row)],
        out_shape=[jax.ShapeDtypeStruct((N, ACT), jnp.float32)],
    )(m0, m1, nf, we2at, we2bt, be2r, wnat, wnbt, bnr, wn2t, bn2r,
      wrot, bror, wro2t, bro2r)


def kernel(obs, h0, c0, edge_index, W1a, b1a, W1b, b1b, W_ih, b_ih, W_hh, b_hh,
           Wm1, bm1, We, be, We2, be2, Wn, bn, Wn2, bn2, Wu, bu, Wu2, bu2,
           Wro, bro, Wro2, bro2):
    h = h0[0]
    c = c0[0]
    w1at = W1a.T
    w1bt = W1b.T
    wiht = W_ih.T
    whht = W_hh.T
    bg = (b_ih + b_hh)[None, :]
    wm1t = Wm1.T
    wst = We[:, :OUT].T
    wdt = We[:, OUT:].T

    h_new, c_new, nf, a_ext, b_ext = _tc1(
        obs, h, c, w1at, b1a[None, :], w1bt, b1b[None, :], wiht, whht, bg,
        wm1t, bm1[None, :], wst, wdt, be[None, :])

    zrow = jnp.zeros((8, WT), jnp.float32)
    a_tab = jnp.concatenate([a_ext.reshape(NC * N, WT), zrow], axis=0)
    b_tab = jnp.concatenate([b_ext.reshape(NC * N, WT), zrow], axis=0)
    src = edge_index[0]
    dst = edge_index[1]
    padv = jnp.full((EPAD,), ZROW, jnp.int32)
    s0 = jnp.concatenate([src, padv]).reshape(EP // K, K)
    d0 = jnp.concatenate([dst, padv]).reshape(EP // K, K)
    s1 = jnp.concatenate([src + N, padv]).reshape(EP // K, K)
    d1 = jnp.concatenate([dst + N, padv]).reshape(EP // K, K)
    g0 = jnp.stack([s0, d0], axis=1)
    g1 = jnp.stack([s1, d1], axis=1)
    gidx = jnp.stack([g0, g1], axis=0).reshape(-1)
    dstp = jnp.concatenate([dst, jnp.full((EPAD,), DROW, jnp.int32)])
    zeros_hbm = jnp.zeros((NP, W), jnp.float32)

    m = _sc_edge(a_tab, b_tab, gidx, dstp, zeros_hbm)

    p0 = jnp.zeros((W, LAST), jnp.float32).at[:128].set(We2[:, :LAST].T)
    p1 = jnp.zeros((W, LAST), jnp.float32).at[:128].set(We2[:, LAST:].T)

    q = _tc2(m[:N], m[NP:NP + N], nf,
             p0, p1, be2[None, :],
             Wn[:, :OUT].T, Wn[:, OUT:].T, bn[None, :],
             Wn2.T, bn2[None, :], Wro.T, bro[None, :], Wro2.T, bro2[None, :])[0]

    return q, h_new[None], c_new[None]


# speedup vs baseline: 1.1613x; 1.0013x over previous
import functools

import jax
import jax.numpy as jnp
from jax import lax
from jax.experimental import pallas as pl
from jax.experimental.pallas import tpu as pltpu
from jax.experimental.pallas import tpu_sc as plsc

N = 10000
E = 320000
D_OBS = 128
HID = 256
MID = 128
LSTM_D = 128
OUT = 128
RFM = 256
LAST = 128
LH = 128
ACT = 16

W = 144
WT = 144
NS = 16
NC = 2
K = 128
EPAD = 3584
EP = E + EPAD
EPT = EP // NS
NCHUNK = EPT // K
ZROW = NC * N
DROW = N
NP = 10112
RPT = NP // NS

R1 = 2000
R2 = 2000

_HI = jax.lax.Precision.DEFAULT


def _sigmoid(x):
    return 1.0 / (1.0 + jnp.exp(-x))


def _tc1_body(obs, h, c, w1at, b1a, w1bt, b1b, wiht, whht, bg, wm1t, bm1,
              wst, wdt, be_, h_o, c_o, nf_o, a_o, b_o):
    x = jnp.maximum(jnp.dot(obs[...], w1at[...], precision=_HI) + b1a[...], 0.0)
    x = jnp.dot(x, w1bt[...], precision=_HI) + b1b[...]
    g = (jnp.dot(x, wiht[...], precision=_HI)
         + jnp.dot(h[...], whht[...], precision=_HI) + bg[...])
    i_g = _sigmoid(g[:, :LSTM_D])
    f_g = _sigmoid(g[:, LSTM_D:2 * LSTM_D])
    g_g = jnp.tanh(g[:, 2 * LSTM_D:3 * LSTM_D])
    o_g = _sigmoid(g[:, 3 * LSTM_D:])
    c_new = f_g * c[...] + i_g * g_g
    h_new = o_g * jnp.tanh(c_new)
    nf = jnp.dot(jnp.maximum(h_new, 0.0), wm1t[...], precision=_HI) + bm1[...]
    a_full = jnp.dot(nf, wst[...], precision=_HI)
    b_full = jnp.dot(nf, wdt[...], precision=_HI) + be_[...]
    h_o[...] = h_new
    c_o[...] = c_new
    nf_o[...] = nf
    rows = a_full.shape[0]
    colid = lax.broadcasted_iota(jnp.int32, (rows, 16), 1)
    pad_one = jnp.where(colid == 0, 1.0, 0.0).astype(jnp.float32)
    pad_zero = jnp.zeros((rows, 16), jnp.float32)
    a_o[0] = jnp.concatenate([a_full[:, :128], pad_one], axis=1)
    a_o[1] = jnp.concatenate([a_full[:, 128:], pad_one], axis=1)
    b_o[0] = jnp.concatenate([b_full[:, :128], pad_zero], axis=1)
    b_o[1] = jnp.concatenate([b_full[:, 128:], pad_zero], axis=1)


def _tc1(obs, h, c, w1at, b1a, w1bt, b1b, wiht, whht, bg, wm1t, bm1, wst, wdt, be_):
    grid = (N // R1,)
    row = lambda i: (i, 0)
    fixed = lambda i: (0, 0)
    out3 = lambda i: (0, i, 0)
    return pl.pallas_call(
        _tc1_body,
        grid=grid,
        in_specs=[
            pl.BlockSpec((R1, D_OBS), row),
            pl.BlockSpec((R1, LSTM_D), row),
            pl.BlockSpec((R1, LSTM_D), row),
            pl.BlockSpec((D_OBS, HID), fixed),
            pl.BlockSpec((1, HID), fixed),
            pl.BlockSpec((HID, MID), fixed),
            pl.BlockSpec((1, MID), fixed),
            pl.BlockSpec((MID, 4 * LSTM_D), fixed),
            pl.BlockSpec((LSTM_D, 4 * LSTM_D), fixed),
            pl.BlockSpec((1, 4 * LSTM_D), fixed),
            pl.BlockSpec((LSTM_D, OUT), fixed),
            pl.BlockSpec((1, OUT), fixed),
            pl.BlockSpec((OUT, RFM), fixed),
            pl.BlockSpec((OUT, RFM), fixed),
            pl.BlockSpec((1, RFM), fixed),
        ],
        out_specs=[
            pl.BlockSpec((R1, LSTM_D), row),
            pl.BlockSpec((R1, LSTM_D), row),
            pl.BlockSpec((R1, OUT), row),
            pl.BlockSpec((NC, R1, WT), out3),
            pl.BlockSpec((NC, R1, WT), out3),
        ],
        out_shape=[
            jax.ShapeDtypeStruct((N, LSTM_D), jnp.float32),
            jax.ShapeDtypeStruct((N, LSTM_D), jnp.float32),
            jax.ShapeDtypeStruct((N, OUT), jnp.float32),
            jax.ShapeDtypeStruct((NC, N, WT), jnp.float32),
            jax.ShapeDtypeStruct((NC, N, WT), jnp.float32),
        ],
    )(obs, h, c, w1at, b1a, w1bt, b1b, wiht, whht, bg, wm1t, bm1, wst, wdt, be_)


def _sc_body(a_tab, b_tab, gidx, dst1, zeros_hbm, m_out,
             gbuf0, gbuf1, sbuf0, sbuf1, abuf0, abuf1, acc,
             semg0, semg1, sems0, sems1, semi):
    c = lax.axis_index("c")
    s = lax.axis_index("s")
    rb = s * RPT
    pltpu.sync_copy(zeros_hbm.at[pl.ds(rb, RPT)], acc.at[pl.ds(rb, RPT)])
    plsc.subcore_barrier()

    GBUF = [gbuf0, gbuf1]
    SBUF = [sbuf0, sbuf1]
    ABUF = [abuf0, abuf1]
    SEMG = [semg0, semg1]
    SEMS = [sems0, sems1]

    gbase = c * (NS * NCHUNK) + s * NCHUNK
    ebase = s * EPT

    def fetch_idx(k, slot):
        pltpu.async_copy(gidx.at[pl.ds((gbase + k) * 2 * K, 2 * K)],
                         GBUF[slot], semi)
        pltpu.async_copy(dst1.at[pl.ds(ebase + k * K, K)], SBUF[slot], semi)

    def wait_idx(slot):
        pltpu.make_async_copy(gidx.at[pl.ds(0, 2 * K)], GBUF[slot], semi).wait()
        pltpu.make_async_copy(dst1.at[pl.ds(0, K)], SBUF[slot], semi).wait()

    def start_a(slot):
        pltpu.async_copy(a_tab.at[GBUF[slot].at[pl.ds(0, K)]], ABUF[slot],
                         SEMG[slot])

    def start_b_add(slot):
        pltpu.async_copy(b_tab.at[GBUF[slot].at[pl.ds(K, K)]], ABUF[slot],
                         SEMG[slot], add=True)

    def wait_g(slot):
        pltpu.make_async_copy(a_tab.at[pl.ds(0, K)], ABUF[slot],
                              SEMG[slot]).wait()

    fetch_idx(0, 0)
    wait_idx(0)
    start_a(0)

    @pl.loop(0, NCHUNK // 2)
    def _pair(i):
        for b in range(2):
            p, q = b, 1 - b
            k = 2 * i + b

            wait_g(p)
            start_b_add(p)

            def drain_q():
                pltpu.make_async_copy(ABUF[q], acc.at[pl.ds(0, K)],
                                      SEMS[q]).wait()

            def prefetch():
                fetch_idx(k + 1, q)
                wait_idx(q)
                start_a(q)

            if b == 0:
                pl.when(i > 0)(drain_q)
                prefetch()
            else:
                drain_q()
                pl.when(i < NCHUNK // 2 - 1)(prefetch)

            wait_g(p)

            @pl.loop(0, K, unroll=2)
            def _relu(r):
                for j in range(WT // 16):
                    v = ABUF[p][r, pl.ds(j * 16, 16)]
                    ABUF[p][r, pl.ds(j * 16, 16)] = jnp.maximum(v, 0.0)

            pltpu.async_copy(ABUF[p], acc.at[SBUF[p]], SEMS[p], add=True)

    pltpu.make_async_copy(ABUF[1], acc.at[pl.ds(0, K)], SEMS[1]).wait()
    plsc.subcore_barrier()
    pltpu.sync_copy(acc.at[pl.ds(rb, RPT)], m_out.at[pl.ds(c * NP + rb, RPT)])


def _sc_edge(a_tab, b_tab, gidx, dst1, zeros_hbm):
    mesh = plsc.VectorSubcoreMesh(core_axis_name="c", subcore_axis_name="s")
    f = pl.kernel(
        _sc_body,
        out_type=jax.ShapeDtypeStruct((NC * NP, W), jnp.float32),
        mesh=mesh,
        scratch_types=[
            pltpu.VMEM((2 * K,), jnp.int32),
            pltpu.VMEM((2 * K,), jnp.int32),
            pltpu.VMEM((K,), jnp.int32),
            pltpu.VMEM((K,), jnp.int32),
            pltpu.VMEM((K, WT), jnp.float32),
            pltpu.VMEM((K, WT), jnp.float32),
            pltpu.VMEM_SHARED((NP, W), jnp.float32),
            pltpu.SemaphoreType.DMA,
            pltpu.SemaphoreType.DMA,
            pltpu.SemaphoreType.DMA,
            pltpu.SemaphoreType.DMA,
            pltpu.SemaphoreType.DMA,
        ],
        compiler_params=pltpu.CompilerParams(use_tc_tiling_on_sc=False),
    )
    return f(a_tab, b_tab, gidx, dst1, zeros_hbm)


def _tc2_body(m0, m1, nf, we2at, we2bt, be2r, wnat, wnbt, bnr, wn2t, bn2r,
              wrot, bror, wro2t, bro2r, q_o):
    h_agg = (jnp.dot(m0[...], we2at[...], precision=_HI)
             + jnp.dot(m1[...], we2bt[...], precision=_HI)
             + m0[:, 128:129] * be2r[...])
    nh = jnp.maximum(jnp.dot(nf[...], wnat[...], precision=_HI)
                     + jnp.dot(h_agg, wnbt[...], precision=_HI) + bnr[...], 0.0)
    node_out = jnp.dot(nh, wn2t[...], precision=_HI) + bn2r[...]
    qh = jnp.maximum(jnp.dot(node_out, wrot[...], precision=_HI) + bror[...], 0.0)
    q_o[...] = jnp.dot(qh, wro2t[...], precision=_HI) + bro2r[...]


def _tc2(m0, m1, nf, we2at, we2bt, be2r, wnat, wnbt, bnr, wn2t, bn2r,
         wrot, bror, wro2t, bro2r):
    grid = (N // R2,)
    row = lambda i: (i, 0)
    fixed = lambda i: (0, 0)
    return pl.pallas_call(
        _tc2_body,
        grid=grid,
        in_specs=[
            pl.BlockSpec((R2, W), row),
            pl.BlockSpec((R2, W), row),
            pl.BlockSpec((R2, OUT), row),
            pl.BlockSpec((W, LAST), fixed),
            pl.BlockSpec((W, LAST), fixed),
            pl.BlockSpec((1, LAST), fixed),
            pl.BlockSpec((OUT, RFM), fixed),
            pl.BlockSpec((LAST, RFM), fixed),
            pl.BlockSpec((1, RFM), fixed),
            pl.BlockSpec((RFM, LAST), fixed),
            pl.BlockSpec((1, LAST), fixed),
            pl.BlockSpec((LAST, LH), fixed),
            pl.BlockSpec((1, LH), fixed),
            pl.BlockSpec((LH, ACT), fixed),
            pl.BlockSpec((1, ACT), fixed),
        ],
        out_specs=[pl.BlockSpec((R2, ACT), row)],
        out_shape=[jax.ShapeDtypeStruct((N, ACT), jnp.float32)],
    )(m0, m1, nf, we2at, we2bt, be2r, wnat, wnbt, bnr, wn2t, bn2r,
      wrot, bror, wro2t, bro2r)


def kernel(obs, h0, c0, edge_index, W1a, b1a, W1b, b1b, W_ih, b_ih, W_hh, b_hh,
           Wm1, bm1, We, be, We2, be2, Wn, bn, Wn2, bn2, Wu, bu, Wu2, bu2,
           Wro, bro, Wro2, bro2):
    h = h0[0]
    c = c0[0]
    w1at = W1a.T
    w1bt = W1b.T
    wiht = W_ih.T
    whht = W_hh.T
    bg = (b_ih + b_hh)[None, :]
    wm1t = Wm1.T
    wst = We[:, :OUT].T
    wdt = We[:, OUT:].T

    h_new, c_new, nf, a_ext, b_ext = _tc1(
        obs, h, c, w1at, b1a[None, :], w1bt, b1b[None, :], wiht, whht, bg,
        wm1t, bm1[None, :], wst, wdt, be[None, :])

    zrow = jnp.zeros((8, WT), jnp.float32)
    a_tab = jnp.concatenate([a_ext.reshape(NC * N, WT), zrow], axis=0)
    b_tab = jnp.concatenate([b_ext.reshape(NC * N, WT), zrow], axis=0)
    src = edge_index[0]
    dst = edge_index[1]
    padv = jnp.full((EPAD,), ZROW, jnp.int32)
    s0 = jnp.concatenate([src, padv]).reshape(EP // K, K)
    d0 = jnp.concatenate([dst, padv]).reshape(EP // K, K)
    s1 = jnp.concatenate([src + N, padv]).reshape(EP // K, K)
    d1 = jnp.concatenate([dst + N, padv]).reshape(EP // K, K)
    g0 = jnp.stack([s0, d0], axis=1)
    g1 = jnp.stack([s1, d1], axis=1)
    gidx = jnp.stack([g0, g1], axis=0).reshape(-1)
    dstp = jnp.concatenate(
        [dst, (jnp.arange(EPAD, dtype=jnp.int32) * 8) % N])
    zeros_hbm = jnp.zeros((NP, W), jnp.float32)

    m = _sc_edge(a_tab, b_tab, gidx, dstp, zeros_hbm)

    p0 = jnp.zeros((W, LAST), jnp.float32).at[:128].set(We2[:, :LAST].T)
    p1 = jnp.zeros((W, LAST), jnp.float32).at[:128].set(We2[:, LAST:].T)

    q = _tc2(m[:N], m[NP:NP + N], nf,
             p0, p1, be2[None, :],
             Wn[:, :OUT].T, Wn[:, OUT:].T, bn[None, :],
             Wn2.T, bn2[None, :], Wro.T, bro[None, :], Wro2.T, bro2[None, :])[0]

    return q, h_new[None], c_new[None]


# speedup vs baseline: 1.3651x; 1.1755x over previous
import functools

import jax
import jax.numpy as jnp
from jax import lax
from jax.experimental import pallas as pl
from jax.experimental.pallas import tpu as pltpu
from jax.experimental.pallas import tpu_sc as plsc

N = 10000
E = 320000
D_OBS = 128
HID = 256
MID = 128
LSTM_D = 128
OUT = 128
RFM = 256
LAST = 128
LH = 128
ACT = 16

W = 144
WT = 144
NS = 16
NC = 2
K = 80
EPAD = 0
EP = E + EPAD
EPT = EP // NS
NCHUNK = EPT // K
ZROW = NC * N
DROW = N
NP = 10112
RPT = NP // NS

R1 = 2000
R2 = 2000

_HI = jax.lax.Precision.DEFAULT


def _sigmoid(x):
    return 1.0 / (1.0 + jnp.exp(-x))


def _tc1_body(obs, h, c, w1at, b1a, w1bt, b1b, wiht, whht, bg, wm1t, bm1,
              wst, wdt, be_, h_o, c_o, nf_o, a_o, b_o):
    x = jnp.maximum(jnp.dot(obs[...], w1at[...], precision=_HI) + b1a[...], 0.0)
    x = jnp.dot(x, w1bt[...], precision=_HI) + b1b[...]
    g = (jnp.dot(x, wiht[...], precision=_HI)
         + jnp.dot(h[...], whht[...], precision=_HI) + bg[...])
    i_g = _sigmoid(g[:, :LSTM_D])
    f_g = _sigmoid(g[:, LSTM_D:2 * LSTM_D])
    g_g = jnp.tanh(g[:, 2 * LSTM_D:3 * LSTM_D])
    o_g = _sigmoid(g[:, 3 * LSTM_D:])
    c_new = f_g * c[...] + i_g * g_g
    h_new = o_g * jnp.tanh(c_new)
    nf = jnp.dot(jnp.maximum(h_new, 0.0), wm1t[...], precision=_HI) + bm1[...]
    a_full = jnp.dot(nf, wst[...], precision=_HI)
    b_full = jnp.dot(nf, wdt[...], precision=_HI) + be_[...]
    h_o[...] = h_new
    c_o[...] = c_new
    nf_o[...] = nf
    rows = a_full.shape[0]
    colid = lax.broadcasted_iota(jnp.int32, (rows, 16), 1)
    pad_one = jnp.where(colid == 0, 1.0, 0.0).astype(jnp.float32)
    pad_zero = jnp.zeros((rows, 16), jnp.float32)
    a_o[0] = jnp.concatenate([a_full[:, :128], pad_one], axis=1)
    a_o[1] = jnp.concatenate([a_full[:, 128:], pad_one], axis=1)
    b_o[0] = jnp.concatenate([b_full[:, :128], pad_zero], axis=1)
    b_o[1] = jnp.concatenate([b_full[:, 128:], pad_zero], axis=1)


def _tc1(obs, h, c, w1at, b1a, w1bt, b1b, wiht, whht, bg, wm1t, bm1, wst, wdt, be_):
    grid = (N // R1,)
    row = lambda i: (i, 0)
    fixed = lambda i: (0, 0)
    out3 = lambda i: (0, i, 0)
    return pl.pallas_call(
        _tc1_body,
        grid=grid,
        in_specs=[
            pl.BlockSpec((R1, D_OBS), row),
            pl.BlockSpec((R1, LSTM_D), row),
            pl.BlockSpec((R1, LSTM_D), row),
            pl.BlockSpec((D_OBS, HID), fixed),
            pl.BlockSpec((1, HID), fixed),
            pl.BlockSpec((HID, MID), fixed),
            pl.BlockSpec((1, MID), fixed),
            pl.BlockSpec((MID, 4 * LSTM_D), fixed),
            pl.BlockSpec((LSTM_D, 4 * LSTM_D), fixed),
            pl.BlockSpec((1, 4 * LSTM_D), fixed),
            pl.BlockSpec((LSTM_D, OUT), fixed),
            pl.BlockSpec((1, OUT), fixed),
            pl.BlockSpec((OUT, RFM), fixed),
            pl.BlockSpec((OUT, RFM), fixed),
            pl.BlockSpec((1, RFM), fixed),
        ],
        out_specs=[
            pl.BlockSpec((R1, LSTM_D), row),
            pl.BlockSpec((R1, LSTM_D), row),
            pl.BlockSpec((R1, OUT), row),
            pl.BlockSpec((NC, R1, WT), out3),
            pl.BlockSpec((NC, R1, WT), out3),
        ],
        out_shape=[
            jax.ShapeDtypeStruct((N, LSTM_D), jnp.float32),
            jax.ShapeDtypeStruct((N, LSTM_D), jnp.float32),
            jax.ShapeDtypeStruct((N, OUT), jnp.float32),
            jax.ShapeDtypeStruct((NC, N, WT), jnp.float32),
            jax.ShapeDtypeStruct((NC, N, WT), jnp.float32),
        ],
    )(obs, h, c, w1at, b1a, w1bt, b1b, wiht, whht, bg, wm1t, bm1, wst, wdt, be_)


def _sc_body(a_tab, b_tab, gidx, dst1, zeros_hbm, m_out,
             gbuf0, gbuf1, sbuf0, sbuf1, abuf0, abuf1, acc,
             semg0, semg1, sems0, sems1, semi):
    c = lax.axis_index("c")
    s = lax.axis_index("s")
    rb = s * RPT
    pltpu.sync_copy(zeros_hbm.at[pl.ds(rb, RPT)], acc.at[pl.ds(rb, RPT)])
    plsc.subcore_barrier()

    GBUF = [gbuf0, gbuf1]
    SBUF = [sbuf0, sbuf1]
    ABUF = [abuf0, abuf1]
    SEMG = [semg0, semg1]
    SEMS = [sems0, sems1]

    gbase = c * (NS * NCHUNK) + s * NCHUNK
    ebase = s * EPT

    def fetch_idx(k, slot):
        pltpu.async_copy(gidx.at[pl.ds((gbase + k) * 2 * K, 2 * K)],
                         GBUF[slot], semi)
        pltpu.async_copy(dst1.at[pl.ds(ebase + k * K, K)], SBUF[slot], semi)

    def wait_idx(slot):
        pltpu.make_async_copy(gidx.at[pl.ds(0, 2 * K)], GBUF[slot], semi).wait()
        pltpu.make_async_copy(dst1.at[pl.ds(0, K)], SBUF[slot], semi).wait()

    def start_a(slot):
        pltpu.async_copy(a_tab.at[GBUF[slot].at[pl.ds(0, K)]], ABUF[slot],
                         SEMG[slot])

    def start_b_add(slot):
        pltpu.async_copy(b_tab.at[GBUF[slot].at[pl.ds(K, K)]], ABUF[slot],
                         SEMG[slot], add=True)

    def wait_g(slot):
        pltpu.make_async_copy(a_tab.at[pl.ds(0, K)], ABUF[slot],
                              SEMG[slot]).wait()

    fetch_idx(0, 0)
    wait_idx(0)
    start_a(0)

    @pl.loop(0, NCHUNK // 2)
    def _pair(i):
        for b in range(2):
            p, q = b, 1 - b
            k = 2 * i + b

            wait_g(p)
            start_b_add(p)

            def drain_q():
                pltpu.make_async_copy(ABUF[q], acc.at[pl.ds(0, K)],
                                      SEMS[q]).wait()

            def prefetch():
                fetch_idx(k + 1, q)
                wait_idx(q)
                start_a(q)

            if b == 0:
                pl.when(i > 0)(drain_q)
                prefetch()
            else:
                drain_q()
                pl.when(i < NCHUNK // 2 - 1)(prefetch)

            wait_g(p)

            @pl.loop(0, K, unroll=2)
            def _relu(r):
                for j in range(128 // 16):
                    v = ABUF[p][r, pl.ds(j * 16, 16)]
                    ABUF[p][r, pl.ds(j * 16, 16)] = jnp.maximum(v, 0.0)

            pltpu.async_copy(ABUF[p], acc.at[SBUF[p]], SEMS[p], add=True)

    pltpu.make_async_copy(ABUF[1], acc.at[pl.ds(0, K)], SEMS[1]).wait()
    plsc.subcore_barrier()
    pltpu.sync_copy(acc.at[pl.ds(rb, RPT)], m_out.at[pl.ds(c * NP + rb, RPT)])


def _sc_edge(a_tab, b_tab, gidx, dst1, zeros_hbm):
    mesh = plsc.VectorSubcoreMesh(core_axis_name="c", subcore_axis_name="s")
    f = pl.kernel(
        _sc_body,
        out_type=jax.ShapeDtypeStruct((NC * NP, W), jnp.float32),
        mesh=mesh,
        scratch_types=[
            pltpu.VMEM((2 * K,), jnp.int32),
            pltpu.VMEM((2 * K,), jnp.int32),
            pltpu.VMEM((K,), jnp.int32),
            pltpu.VMEM((K,), jnp.int32),
            pltpu.VMEM((K, WT), jnp.float32),
            pltpu.VMEM((K, WT), jnp.float32),
            pltpu.VMEM_SHARED((NP, W), jnp.float32),
            pltpu.SemaphoreType.DMA,
            pltpu.SemaphoreType.DMA,
            pltpu.SemaphoreType.DMA,
            pltpu.SemaphoreType.DMA,
            pltpu.SemaphoreType.DMA,
        ],
        compiler_params=pltpu.CompilerParams(use_tc_tiling_on_sc=False),
    )
    return f(a_tab, b_tab, gidx, dst1, zeros_hbm)


def _tc2_body(m0, m1, nf, we2at, we2bt, be2r, wnat, wnbt, bnr, wn2t, bn2r,
              wrot, bror, wro2t, bro2r, q_o):
    h_agg = (jnp.dot(m0[...], we2at[...], precision=_HI)
             + jnp.dot(m1[...], we2bt[...], precision=_HI)
             + m0[:, 128:129] * be2r[...])
    nh = jnp.maximum(jnp.dot(nf[...], wnat[...], precision=_HI)
                     + jnp.dot(h_agg, wnbt[...], precision=_HI) + bnr[...], 0.0)
    node_out = jnp.dot(nh, wn2t[...], precision=_HI) + bn2r[...]
    qh = jnp.maximum(jnp.dot(node_out, wrot[...], precision=_HI) + bror[...], 0.0)
    q_o[...] = jnp.dot(qh, wro2t[...], precision=_HI) + bro2r[...]


def _tc2(m0, m1, nf, we2at, we2bt, be2r, wnat, wnbt, bnr, wn2t, bn2r,
         wrot, bror, wro2t, bro2r):
    grid = (N // R2,)
    row = lambda i: (i, 0)
    fixed = lambda i: (0, 0)
    return pl.pallas_call(
        _tc2_body,
        grid=grid,
        in_specs=[
            pl.BlockSpec((R2, W), row),
            pl.BlockSpec((R2, W), row),
            pl.BlockSpec((R2, OUT), row),
            pl.BlockSpec((W, LAST), fixed),
            pl.BlockSpec((W, LAST), fixed),
            pl.BlockSpec((1, LAST), fixed),
            pl.BlockSpec((OUT, RFM), fixed),
            pl.BlockSpec((LAST, RFM), fixed),
            pl.BlockSpec((1, RFM), fixed),
            pl.BlockSpec((RFM, LAST), fixed),
            pl.BlockSpec((1, LAST), fixed),
            pl.BlockSpec((LAST, LH), fixed),
            pl.BlockSpec((1, LH), fixed),
            pl.BlockSpec((LH, ACT), fixed),
            pl.BlockSpec((1, ACT), fixed),
        ],
        out_specs=[pl.BlockSpec((R2, ACT), row)],
        out_shape=[jax.ShapeDtypeStruct((N, ACT), jnp.float32)],
    )(m0, m1, nf, we2at, we2bt, be2r, wnat, wnbt, bnr, wn2t, bn2r,
      wrot, bror, wro2t, bro2r)


def kernel(obs, h0, c0, edge_index, W1a, b1a, W1b, b1b, W_ih, b_ih, W_hh, b_hh,
           Wm1, bm1, We, be, We2, be2, Wn, bn, Wn2, bn2, Wu, bu, Wu2, bu2,
           Wro, bro, Wro2, bro2):
    h = h0[0]
    c = c0[0]
    w1at = W1a.T
    w1bt = W1b.T
    wiht = W_ih.T
    whht = W_hh.T
    bg = (b_ih + b_hh)[None, :]
    wm1t = Wm1.T
    wst = We[:, :OUT].T
    wdt = We[:, OUT:].T

    h_new, c_new, nf, a_ext, b_ext = _tc1(
        obs, h, c, w1at, b1a[None, :], w1bt, b1b[None, :], wiht, whht, bg,
        wm1t, bm1[None, :], wst, wdt, be[None, :])

    zrow = jnp.zeros((8, WT), jnp.float32)
    a_tab = jnp.concatenate([a_ext.reshape(NC * N, WT), zrow], axis=0)
    b_tab = jnp.concatenate([b_ext.reshape(NC * N, WT), zrow], axis=0)
    src = edge_index[0]
    dst = edge_index[1]
    padv = jnp.full((EPAD,), ZROW, jnp.int32)
    s0 = jnp.concatenate([src, padv]).reshape(EP // K, K)
    d0 = jnp.concatenate([dst, padv]).reshape(EP // K, K)
    s1 = jnp.concatenate([src + N, padv]).reshape(EP // K, K)
    d1 = jnp.concatenate([dst + N, padv]).reshape(EP // K, K)
    g0 = jnp.stack([s0, d0], axis=1)
    g1 = jnp.stack([s1, d1], axis=1)
    gidx = jnp.stack([g0, g1], axis=0).reshape(-1)
    dstp = jnp.concatenate(
        [dst, (jnp.arange(EPAD, dtype=jnp.int32) * 8) % N])
    zeros_hbm = jnp.zeros((NP, W), jnp.float32)

    m = _sc_edge(a_tab, b_tab, gidx, dstp, zeros_hbm)

    p0 = jnp.zeros((W, LAST), jnp.float32).at[:128].set(We2[:, :LAST].T)
    p1 = jnp.zeros((W, LAST), jnp.float32).at[:128].set(We2[:, LAST:].T)

    q = _tc2(m[:N], m[NP:NP + N], nf,
             p0, p1, be2[None, :],
             Wn[:, :OUT].T, Wn[:, OUT:].T, bn[None, :],
             Wn2.T, bn2[None, :], Wro.T, bro[None, :], Wro2.T, bro2[None, :])[0]

    return q, h_new[None], c_new[None]


# speedup vs baseline: 1.4269x; 1.0453x over previous
import functools

import jax
import jax.numpy as jnp
from jax import lax
from jax.experimental import pallas as pl
from jax.experimental.pallas import tpu as pltpu
from jax.experimental.pallas import tpu_sc as plsc

N = 10000
E = 320000
D_OBS = 128
HID = 256
MID = 128
LSTM_D = 128
OUT = 128
RFM = 256
LAST = 128
LH = 128
ACT = 16

W = 144
WT = 144
NS = 16
NC = 2
K = 80
EPT = E // NS
NCHUNK = EPT // K
NP = 10240
RPT = NP // NS

R1 = 2000
R2 = 2000

_HI = jax.lax.Precision.DEFAULT


def _sigmoid(x):
    return 1.0 / (1.0 + jnp.exp(-x))


def _tc1_body(obs, h, c, w1at, b1a, w1bt, b1b, wiht, whht, bg, wm1t, bm1,
              wst, wdt, be_, h_o, c_o, nf_o, a_o, b_o):
    x = jnp.maximum(jnp.dot(obs[...], w1at[...], precision=_HI) + b1a[...], 0.0)
    x = jnp.dot(x, w1bt[...], precision=_HI) + b1b[...]
    g = (jnp.dot(x, wiht[...], precision=_HI)
         + jnp.dot(h[...], whht[...], precision=_HI) + bg[...])
    i_g = _sigmoid(g[:, :LSTM_D])
    f_g = _sigmoid(g[:, LSTM_D:2 * LSTM_D])
    g_g = jnp.tanh(g[:, 2 * LSTM_D:3 * LSTM_D])
    o_g = _sigmoid(g[:, 3 * LSTM_D:])
    c_new = f_g * c[...] + i_g * g_g
    h_new = o_g * jnp.tanh(c_new)
    nf = jnp.dot(jnp.maximum(h_new, 0.0), wm1t[...], precision=_HI) + bm1[...]
    a_full = jnp.dot(nf, wst[...], precision=_HI)
    b_full = jnp.dot(nf, wdt[...], precision=_HI) + be_[...]
    h_o[...] = h_new
    c_o[...] = c_new
    nf_o[...] = nf
    rows = a_full.shape[0]
    colid = lax.broadcasted_iota(jnp.int32, (rows, 16), 1)
    pad_one = jnp.where(colid == 0, 1.0, 0.0).astype(jnp.float32)
    pad_zero = jnp.zeros((rows, 16), jnp.float32)
    a_o[0] = jnp.concatenate([a_full[:, :128], pad_one], axis=1)
    a_o[1] = jnp.concatenate([a_full[:, 128:], pad_one], axis=1)
    b_o[0] = jnp.concatenate([b_full[:, :128], pad_zero], axis=1)
    b_o[1] = jnp.concatenate([b_full[:, 128:], pad_zero], axis=1)


def _tc1(obs, h, c, w1at, b1a, w1bt, b1b, wiht, whht, bg, wm1t, bm1, wst, wdt, be_):
    grid = (N // R1,)
    row = lambda i: (i, 0)
    fixed = lambda i: (0, 0)
    out3 = lambda i: (0, i, 0)
    return pl.pallas_call(
        _tc1_body,
        grid=grid,
        in_specs=[
            pl.BlockSpec((R1, D_OBS), row),
            pl.BlockSpec((R1, LSTM_D), row),
            pl.BlockSpec((R1, LSTM_D), row),
            pl.BlockSpec((D_OBS, HID), fixed),
            pl.BlockSpec((1, HID), fixed),
            pl.BlockSpec((HID, MID), fixed),
            pl.BlockSpec((1, MID), fixed),
            pl.BlockSpec((MID, 4 * LSTM_D), fixed),
            pl.BlockSpec((LSTM_D, 4 * LSTM_D), fixed),
            pl.BlockSpec((1, 4 * LSTM_D), fixed),
            pl.BlockSpec((LSTM_D, OUT), fixed),
            pl.BlockSpec((1, OUT), fixed),
            pl.BlockSpec((OUT, RFM), fixed),
            pl.BlockSpec((OUT, RFM), fixed),
            pl.BlockSpec((1, RFM), fixed),
        ],
        out_specs=[
            pl.BlockSpec((R1, LSTM_D), row),
            pl.BlockSpec((R1, LSTM_D), row),
            pl.BlockSpec((R1, OUT), row),
            pl.BlockSpec((NC, R1, WT), out3),
            pl.BlockSpec((NC, R1, WT), out3),
        ],
        out_shape=[
            jax.ShapeDtypeStruct((N, LSTM_D), jnp.float32),
            jax.ShapeDtypeStruct((N, LSTM_D), jnp.float32),
            jax.ShapeDtypeStruct((N, OUT), jnp.float32),
            jax.ShapeDtypeStruct((NC, N, WT), jnp.float32),
            jax.ShapeDtypeStruct((NC, N, WT), jnp.float32),
        ],
    )(obs, h, c, w1at, b1a, w1bt, b1b, wiht, whht, bg, wm1t, bm1, wst, wdt, be_)


def _sc_body(a_tab, b_tab, gidx, dst1, zeros_hbm, m_out,
             gbuf0, gbuf1, sbuf0, sbuf1, abuf0, abuf1, acc,
             semg0, semg1, sems0, sems1, semi):
    c = lax.axis_index("c")
    s = lax.axis_index("s")
    rb = s * RPT
    pltpu.sync_copy(zeros_hbm.at[pl.ds(rb, RPT)], acc.at[pl.ds(rb, RPT)])
    plsc.subcore_barrier()

    GBUF = [gbuf0, gbuf1]
    SBUF = [sbuf0, sbuf1]
    ABUF = [abuf0, abuf1]
    SEMG = [semg0, semg1]
    SEMS = [sems0, sems1]

    gbase = c * (NS * NCHUNK) + s * NCHUNK
    ebase = s * EPT

    def fetch_idx(k, slot):
        pltpu.async_copy(gidx.at[pl.ds((gbase + k) * 2 * K, 2 * K)],
                         GBUF[slot], semi)
        pltpu.async_copy(dst1.at[pl.ds(ebase + k * K, K)], SBUF[slot], semi)

    def wait_idx(slot):
        pltpu.make_async_copy(gidx.at[pl.ds(0, 2 * K)], GBUF[slot], semi).wait()
        pltpu.make_async_copy(dst1.at[pl.ds(0, K)], SBUF[slot], semi).wait()

    def start_a(slot):
        pltpu.async_copy(a_tab.at[GBUF[slot].at[pl.ds(0, K)]], ABUF[slot],
                         SEMG[slot])

    def start_b_add(slot):
        pltpu.async_copy(b_tab.at[GBUF[slot].at[pl.ds(K, K)]], ABUF[slot],
                         SEMG[slot], add=True)

    def wait_g(slot):
        pltpu.make_async_copy(a_tab.at[pl.ds(0, K)], ABUF[slot],
                              SEMG[slot]).wait()

    fetch_idx(0, 0)
    wait_idx(0)
    start_a(0)

    @pl.loop(0, NCHUNK // 2)
    def _pair(i):
        for b in range(2):
            p, q = b, 1 - b
            k = 2 * i + b

            wait_g(p)
            start_b_add(p)

            def drain_q():
                pltpu.make_async_copy(ABUF[q], acc.at[pl.ds(0, K)],
                                      SEMS[q]).wait()

            def prefetch():
                fetch_idx(k + 1, q)
                wait_idx(q)
                start_a(q)

            if b == 0:
                pl.when(i > 0)(drain_q)
                prefetch()
            else:
                drain_q()
                pl.when(i < NCHUNK // 2 - 1)(prefetch)

            wait_g(p)

            @pl.loop(0, K, unroll=2)
            def _relu(r):
                for j in range(128 // 16):
                    v = ABUF[p][r, pl.ds(j * 16, 16)]
                    ABUF[p][r, pl.ds(j * 16, 16)] = jnp.maximum(v, 0.0)

            pltpu.async_copy(ABUF[p], acc.at[SBUF[p]], SEMS[p], add=True)

    pltpu.make_async_copy(ABUF[1], acc.at[pl.ds(0, K)], SEMS[1]).wait()
    plsc.subcore_barrier()
    pltpu.sync_copy(acc.at[pl.ds(rb, RPT)], m_out.at[pl.ds(c * NP + rb, RPT)])


def _sc_edge(a_tab, b_tab, gidx, dst1, zeros_hbm):
    mesh = plsc.VectorSubcoreMesh(core_axis_name="c", subcore_axis_name="s")
    f = pl.kernel(
        _sc_body,
        out_type=jax.ShapeDtypeStruct((NC * NP, W), jnp.float32),
        mesh=mesh,
        scratch_types=[
            pltpu.VMEM((2 * K,), jnp.int32),
            pltpu.VMEM((2 * K,), jnp.int32),
            pltpu.VMEM((K,), jnp.int32),
            pltpu.VMEM((K,), jnp.int32),
            pltpu.VMEM((K, WT), jnp.float32),
            pltpu.VMEM((K, WT), jnp.float32),
            pltpu.VMEM_SHARED((NP, W), jnp.float32),
            pltpu.SemaphoreType.DMA,
            pltpu.SemaphoreType.DMA,
            pltpu.SemaphoreType.DMA,
            pltpu.SemaphoreType.DMA,
            pltpu.SemaphoreType.DMA,
        ],
        compiler_params=pltpu.CompilerParams(use_tc_tiling_on_sc=False),
    )
    return f(a_tab, b_tab, gidx, dst1, zeros_hbm)


def _tc2_body(m0, m1, nf, we2at, we2bt, be2r, wnat, wnbt, bnr, wn2t, bn2r,
              wrot, bror, wro2t, bro2r, q_o):
    h_agg = (jnp.dot(m0[...], we2at[...], precision=_HI)
             + jnp.dot(m1[...], we2bt[...], precision=_HI)
             + m0[:, 128:129] * be2r[...])
    nh = jnp.maximum(jnp.dot(nf[...], wnat[...], precision=_HI)
                     + jnp.dot(h_agg, wnbt[...], precision=_HI) + bnr[...], 0.0)
    node_out = jnp.dot(nh, wn2t[...], precision=_HI) + bn2r[...]
    qh = jnp.maximum(jnp.dot(node_out, wrot[...], precision=_HI) + bror[...], 0.0)
    q_o[...] = jnp.dot(qh, wro2t[...], precision=_HI) + bro2r[...]


def _tc2(m0, m1, nf, we2at, we2bt, be2r, wnat, wnbt, bnr, wn2t, bn2r,
         wrot, bror, wro2t, bro2r):
    grid = (N // R2,)
    row = lambda i: (i, 0)
    fixed = lambda i: (0, 0)
    return pl.pallas_call(
        _tc2_body,
        grid=grid,
        in_specs=[
            pl.BlockSpec((R2, W), row),
            pl.BlockSpec((R2, W), row),
            pl.BlockSpec((R2, OUT), row),
            pl.BlockSpec((W, LAST), fixed),
            pl.BlockSpec((W, LAST), fixed),
            pl.BlockSpec((1, LAST), fixed),
            pl.BlockSpec((OUT, RFM), fixed),
            pl.BlockSpec((LAST, RFM), fixed),
            pl.BlockSpec((1, RFM), fixed),
            pl.BlockSpec((RFM, LAST), fixed),
            pl.BlockSpec((1, LAST), fixed),
            pl.BlockSpec((LAST, LH), fixed),
            pl.BlockSpec((1, LH), fixed),
            pl.BlockSpec((LH, ACT), fixed),
            pl.BlockSpec((1, ACT), fixed),
        ],
        out_specs=[pl.BlockSpec((R2, ACT), row)],
        out_shape=[jax.ShapeDtypeStruct((N, ACT), jnp.float32)],
    )(m0, m1, nf, we2at, we2bt, be2r, wnat, wnbt, bnr, wn2t, bn2r,
      wrot, bror, wro2t, bro2r)


def kernel(obs, h0, c0, edge_index, W1a, b1a, W1b, b1b, W_ih, b_ih, W_hh, b_hh,
           Wm1, bm1, We, be, We2, be2, Wn, bn, Wn2, bn2, Wu, bu, Wu2, bu2,
           Wro, bro, Wro2, bro2):
    h = h0[0]
    c = c0[0]
    w1at = W1a.T
    w1bt = W1b.T
    wiht = W_ih.T
    whht = W_hh.T
    bg = (b_ih + b_hh)[None, :]
    wm1t = Wm1.T
    wst = We[:, :OUT].T
    wdt = We[:, OUT:].T

    h_new, c_new, nf, a_ext, b_ext = _tc1(
        obs, h, c, w1at, b1a[None, :], w1bt, b1b[None, :], wiht, whht, bg,
        wm1t, bm1[None, :], wst, wdt, be[None, :])

    a_tab = a_ext.reshape(NC * N, WT)
    b_tab = b_ext.reshape(NC * N, WT)
    src = edge_index[0]
    dst = edge_index[1]
    src_c = src.reshape(E // K, K)
    dst_c = dst.reshape(E // K, K)
    g0 = jnp.stack([src_c, dst_c], axis=1)
    gidx = jnp.stack([g0, g0 + N], axis=0).reshape(-1)
    zeros_hbm = jnp.zeros((NP, W), jnp.float32)

    m = _sc_edge(a_tab, b_tab, gidx, dst, zeros_hbm)

    p0 = jnp.zeros((W, LAST), jnp.float32).at[:128].set(We2[:, :LAST].T)
    p1 = jnp.zeros((W, LAST), jnp.float32).at[:128].set(We2[:, LAST:].T)

    q = _tc2(m[:N], m[NP:NP + N], nf,
             p0, p1, be2[None, :],
             Wn[:, :OUT].T, Wn[:, OUT:].T, bn[None, :],
             Wn2.T, bn2[None, :], Wro.T, bro[None, :], Wro2.T, bro2[None, :])[0]

    return q, h_new[None], c_new[None]
